# Initial kernel scaffold; baseline (speedup 1.0000x reference)
import jax
import jax.numpy as jnp
from jax.experimental import pallas as pl


def kernel(f_atoms, edge_index, graph_ids, W_in, b_in, W1, b1, W2, b2, gamma, beta, eps):
    raise NotImplementedError("write your pallas kernel here")


# speedup vs baseline: 2.8351x; 2.8351x over previous
import functools

import jax
import jax.numpy as jnp
from jax import lax
from jax.experimental import pallas as pl
from jax.experimental.pallas import tpu as pltpu
from jax.experimental.pallas import tpu_sc as plsc

N = 10000
H = 128
E = 320000
G = 256
DEPTH_ = 3
BN_EPS_ = 1e-5

NC = 2
NS = 16
NT = NC * NS
CHUNK = 128
E_PER_TILE = 10240
NCH = E_PER_TILE // CHUNK
E_PAD = NT * E_PER_TILE
AGG_ROWS = 10240
ZROWS = AGG_ROWS // NS
OROWS = 632

RB = 2000
NB = N // RB


_sc_mesh = plsc.VectorSubcoreMesh(
    core_axis_name="c", subcore_axis_name="s", num_cores=NC, num_subcores=NS
)


@functools.partial(
    pl.kernel,
    out_type=jax.ShapeDtypeStruct((NC, N, H), jnp.float32),
    mesh=_sc_mesh,
    scratch_types=[
        pltpu.VMEM((NCH, CHUNK), jnp.int32),
        pltpu.VMEM((NCH, CHUNK), jnp.int32),
        pltpu.VMEM((CHUNK, H), jnp.float32),
        pltpu.VMEM_SHARED((AGG_ROWS, H), jnp.float32),
        pltpu.SemaphoreType.DMA,
    ],
)
def _sc_scatter(x_hbm, src_hbm, tgt_hbm, out_hbm, src_v, tgt_v, rows_v, agg_sh, gsem):
    cid = lax.axis_index("c")
    sid = lax.axis_index("s")
    wid = cid * NS + sid

    zz = jnp.zeros((16,), jnp.float32)

    def _zrow(i, _):
        r = i // 8
        c = (i % 8) * 16
        rows_v[r, pl.ds(c, 16)] = zz
        return 0

    lax.fori_loop(0, CHUNK * 8, _zrow, 0)

    def _zcopy(k, _):
        pltpu.sync_copy(
            rows_v,
            agg_sh.at[pl.ds(sid * ZROWS + k * CHUNK, CHUNK)],
        )
        return 0

    lax.fori_loop(0, ZROWS // CHUNK, _zcopy, 0)
    plsc.subcore_barrier()

    pltpu.sync_copy(src_hbm.at[wid], src_v)
    pltpu.sync_copy(tgt_hbm.at[wid], tgt_v)

    def _edge(j, _):
        pltpu.async_copy(x_hbm.at[src_v.at[j]], rows_v, gsem).wait()
        pltpu.sync_copy(rows_v, agg_sh.at[tgt_v.at[j]], add=True)
        return 0

    lax.fori_loop(0, NCH, _edge, 0)
    plsc.subcore_barrier()

    @pl.when(sid < NS - 1)
    def _():
        pltpu.sync_copy(
            agg_sh.at[pl.ds(sid * OROWS, OROWS)],
            out_hbm.at[cid, pl.ds(sid * OROWS, OROWS)],
        )

    @pl.when(sid == NS - 1)
    def _():
        pltpu.sync_copy(
            agg_sh.at[pl.ds((NS - 1) * OROWS, N - (NS - 1) * OROWS)],
            out_hbm.at[cid, pl.ds((NS - 1) * OROWS, N - (NS - 1) * OROWS)],
        )


def _proj_body(f_ref, w_ref, b_ref, o_ref):
    o_ref[...] = jnp.maximum(
        jnp.dot(f_ref[...], w_ref[...], preferred_element_type=jnp.float32)
        + b_ref[...],
        0.0,
    )


def _proj(f_atoms, W_in, b_in2):
    return pl.pallas_call(
        _proj_body,
        grid=(NB,),
        in_specs=[
            pl.BlockSpec((RB, H), lambda i: (i, 0)),
            pl.BlockSpec((H, H), lambda i: (0, 0)),
            pl.BlockSpec((1, H), lambda i: (0, 0)),
        ],
        out_specs=pl.BlockSpec((RB, H), lambda i: (i, 0)),
        out_shape=jax.ShapeDtypeStruct((N, H), jnp.float32),
    )(f_atoms, W_in, b_in2)


def _mlp_body(a0_ref, a1_ref, x_ref, w1_ref, b1_ref, w2_ref, b2_ref, ep_ref,
              h_ref, st_ref):
    i = pl.program_id(0)
    out = a0_ref[...] + a1_ref[...] + ep_ref[0, 0] * x_ref[...]
    h1 = jnp.maximum(
        jnp.dot(out, w1_ref[...], preferred_element_type=jnp.float32)
        + b1_ref[...],
        0.0,
    )
    h = (
        jnp.dot(h1, w2_ref[...], preferred_element_type=jnp.float32)
        + b2_ref[...]
    )
    h_ref[...] = h

    @pl.when(i == 0)
    def _():
        st_ref[...] = jnp.zeros_like(st_ref)

    st_ref[0:1, :] += jnp.sum(h, axis=0, keepdims=True)
    st_ref[1:2, :] += jnp.sum(h * h, axis=0, keepdims=True)


def _mlp(agg0, agg1, x, W1l, b1l2, W2l, b2l2, ep11):
    return pl.pallas_call(
        _mlp_body,
        grid=(NB,),
        in_specs=[
            pl.BlockSpec((RB, H), lambda i: (i, 0)),
            pl.BlockSpec((RB, H), lambda i: (i, 0)),
            pl.BlockSpec((RB, H), lambda i: (i, 0)),
            pl.BlockSpec((H, H), lambda i: (0, 0)),
            pl.BlockSpec((1, H), lambda i: (0, 0)),
            pl.BlockSpec((H, H), lambda i: (0, 0)),
            pl.BlockSpec((1, H), lambda i: (0, 0)),
            pl.BlockSpec(memory_space=pltpu.SMEM),
        ],
        out_specs=[
            pl.BlockSpec((RB, H), lambda i: (i, 0)),
            pl.BlockSpec((8, H), lambda i: (0, 0)),
        ],
        out_shape=[
            jax.ShapeDtypeStruct((N, H), jnp.float32),
            jax.ShapeDtypeStruct((8, H), jnp.float32),
        ],
    )(agg0, agg1, x, W1l, b1l2, W2l, b2l2, ep11)


def _bn_body(h_ref, st_ref, g_ref, be_ref, x_ref):
    mu = st_ref[0:1, :] * (1.0 / N)
    var = st_ref[1:2, :] * (1.0 / N) - mu * mu
    inv = lax.rsqrt(var + BN_EPS_)
    x_ref[...] = jnp.maximum(
        (h_ref[...] - mu) * (inv * g_ref[...]) + be_ref[...], 0.0
    )


def _bn_relu(h, stats, gl2, bl2):
    return pl.pallas_call(
        _bn_body,
        grid=(NB,),
        in_specs=[
            pl.BlockSpec((RB, H), lambda i: (i, 0)),
            pl.BlockSpec((8, H), lambda i: (0, 0)),
            pl.BlockSpec((1, H), lambda i: (0, 0)),
            pl.BlockSpec((1, H), lambda i: (0, 0)),
        ],
        out_specs=pl.BlockSpec((RB, H), lambda i: (i, 0)),
        out_shape=jax.ShapeDtypeStruct((N, H), jnp.float32),
    )(h, stats, gl2, bl2)


def _pool_body(x_ref, gid_ref, mol_ref, sums_ref, cnts_ref):
    i = pl.program_id(0)

    @pl.when(i == 0)
    def _():
        sums_ref[...] = jnp.zeros_like(sums_ref)
        cnts_ref[...] = jnp.zeros_like(cnts_ref)
        mol_ref[...] = jnp.zeros_like(mol_ref)

    gid = gid_ref[0, 0, :]
    oh = (
        gid[:, None]
        == lax.broadcasted_iota(jnp.int32, (RB, G), 1)
    ).astype(jnp.float32)
    x = x_ref[...]
    sums_ref[...] += lax.dot_general(
        oh, x, (((0,), (0,)), ((), ())), preferred_element_type=jnp.float32
    )
    cnts_ref[...] += lax.dot_general(
        oh, jnp.ones((RB, H), jnp.float32), (((0,), (0,)), ((), ())),
        preferred_element_type=jnp.float32,
    )

    @pl.when(i == NB - 1)
    def _():
        mol_ref[...] = sums_ref[...] / jnp.maximum(cnts_ref[...], 1.0)


def _pool(x, gid3):
    return pl.pallas_call(
        _pool_body,
        grid=(NB,),
        in_specs=[
            pl.BlockSpec((RB, H), lambda i: (i, 0)),
            pl.BlockSpec((1, 1, RB), lambda i: (i, 0, 0)),
        ],
        out_specs=pl.BlockSpec((G, H), lambda i: (0, 0)),
        out_shape=jax.ShapeDtypeStruct((G, H), jnp.float32),
        scratch_shapes=[
            pltpu.VMEM((G, H), jnp.float32),
            pltpu.VMEM((G, H), jnp.float32),
        ],
    )(x, gid3)


def kernel(f_atoms, edge_index, graph_ids, W_in, b_in, W1, b1, W2, b2,
           gamma, beta, eps):
    src = edge_index[0].astype(jnp.int32)
    tgt = edge_index[1].astype(jnp.int32)
    npad = E_PAD - E
    src3 = jnp.concatenate([src, jnp.zeros((npad,), jnp.int32)]).reshape(
        NT, NCH, CHUNK
    )
    tgt3 = jnp.concatenate(
        [tgt, jnp.full((npad,), N, jnp.int32)]
    ).reshape(NT, NCH, CHUNK)
    gid3 = graph_ids.astype(jnp.int32).reshape(NB, 1, RB)

    x = _proj(f_atoms, W_in, b_in.reshape(1, H))
    for l in range(DEPTH_):
        partials = _sc_scatter(x, src3, tgt3)
        h, stats = _mlp(
            partials[0], partials[1], x,
            W1[l], b1[l].reshape(1, H), W2[l], b2[l].reshape(1, H),
            (1.0 + eps[l]).reshape(1, 1),
        )
        x = _bn_relu(h, stats, gamma[l].reshape(1, H), beta[l].reshape(1, H))
    return _pool(x, gid3)


# speedup vs baseline: 3.1942x; 1.1267x over previous
import functools

import jax
import jax.numpy as jnp
from jax import lax
from jax.experimental import pallas as pl
from jax.experimental.pallas import tpu as pltpu
from jax.experimental.pallas import tpu_sc as plsc

N = 10000
H = 128
E = 320000
G = 256
DEPTH_ = 3
BN_EPS_ = 1e-5

NC = 2
NS = 16
NT = NC * NS
CHUNK = 128
E_PER_TILE = 10240
NCH = E_PER_TILE // CHUNK
E_PAD = NT * E_PER_TILE
AGG_ROWS = 10240
ZROWS = AGG_ROWS // NS
OROWS = 632

RB = 2000
NB = N // RB


_sc_mesh = plsc.VectorSubcoreMesh(
    core_axis_name="c", subcore_axis_name="s", num_cores=NC, num_subcores=NS
)


@functools.partial(
    pl.kernel,
    out_type=jax.ShapeDtypeStruct((NC, N, H), jnp.float32),
    mesh=_sc_mesh,
    scratch_types=[
        pltpu.VMEM((NCH, CHUNK), jnp.int32),
        pltpu.VMEM((CHUNK,), jnp.int32),
        pltpu.VMEM((CHUNK,), jnp.int32),
        pltpu.VMEM((CHUNK, H), jnp.float32),
        pltpu.VMEM((CHUNK, H), jnp.float32),
        pltpu.VMEM_SHARED((AGG_ROWS, H), jnp.float32),
        pltpu.SemaphoreType.DMA,
        pltpu.SemaphoreType.DMA,
        pltpu.SemaphoreType.DMA,
        pltpu.SemaphoreType.DMA,
    ],
)
def _sc_scatter(x_hbm, src_hbm, tgt_hbm, out_hbm, src_v, tgt0_v, tgt1_v,
                rows0_v, rows1_v, agg_sh, gsem0, gsem1, tsem0, tsem1):
    cid = lax.axis_index("c")
    sid = lax.axis_index("s")
    wid = cid * NS + sid

    zz = jnp.zeros((16,), jnp.float32)

    def _zrow(i, _):
        r = i // 8
        c = (i % 8) * 16
        rows0_v[r, pl.ds(c, 16)] = zz
        return 0

    lax.fori_loop(0, CHUNK * 8, _zrow, 0)

    def _zcopy(k, _):
        pltpu.sync_copy(
            rows0_v,
            agg_sh.at[pl.ds(sid * ZROWS + k * CHUNK, CHUNK)],
        )
        return 0

    lax.fori_loop(0, ZROWS // CHUNK, _zcopy, 0)
    plsc.subcore_barrier()

    pltpu.sync_copy(src_hbm.at[wid], src_v)
    ebase = wid * E_PER_TILE

    pltpu.async_copy(tgt_hbm.at[pl.ds(ebase, CHUNK)], tgt0_v, tsem0)
    pltpu.async_copy(tgt_hbm.at[pl.ds(ebase + CHUNK, CHUNK)], tgt1_v, tsem1)
    pltpu.async_copy(x_hbm.at[src_v.at[0]], rows0_v, gsem0)
    pltpu.async_copy(x_hbm.at[src_v.at[1]], rows1_v, gsem1)

    @pl.loop(0, NCH, step=2)
    def _edge(j):
        pltpu.make_async_copy(x_hbm.at[src_v.at[j]], rows0_v, gsem0).wait()
        pltpu.make_async_copy(
            tgt_hbm.at[pl.ds(ebase, CHUNK)], tgt0_v, tsem0
        ).wait()
        pltpu.sync_copy(rows0_v, agg_sh.at[tgt0_v], add=True)

        @pl.when(j + 2 < NCH)
        def _():
            pltpu.async_copy(
                tgt_hbm.at[pl.ds(ebase + (j + 2) * CHUNK, CHUNK)],
                tgt0_v, tsem0,
            )
            pltpu.async_copy(x_hbm.at[src_v.at[j + 2]], rows0_v, gsem0)

        pltpu.make_async_copy(
            x_hbm.at[src_v.at[j + 1]], rows1_v, gsem1
        ).wait()
        pltpu.make_async_copy(
            tgt_hbm.at[pl.ds(ebase, CHUNK)], tgt1_v, tsem1
        ).wait()
        pltpu.sync_copy(rows1_v, agg_sh.at[tgt1_v], add=True)

        @pl.when(j + 3 < NCH)
        def _():
            pltpu.async_copy(
                tgt_hbm.at[pl.ds(ebase + (j + 3) * CHUNK, CHUNK)],
                tgt1_v, tsem1,
            )
            pltpu.async_copy(x_hbm.at[src_v.at[j + 3]], rows1_v, gsem1)

    plsc.subcore_barrier()

    @pl.when(sid < NS - 1)
    def _():
        pltpu.sync_copy(
            agg_sh.at[pl.ds(sid * OROWS, OROWS)],
            out_hbm.at[cid, pl.ds(sid * OROWS, OROWS)],
        )

    @pl.when(sid == NS - 1)
    def _():
        pltpu.sync_copy(
            agg_sh.at[pl.ds((NS - 1) * OROWS, N - (NS - 1) * OROWS)],
            out_hbm.at[cid, pl.ds((NS - 1) * OROWS, N - (NS - 1) * OROWS)],
        )


def _proj_body(f_ref, w_ref, b_ref, o_ref):
    o_ref[...] = jnp.maximum(
        jnp.dot(f_ref[...], w_ref[...], preferred_element_type=jnp.float32)
        + b_ref[...],
        0.0,
    )


def _proj(f_atoms, W_in, b_in2):
    return pl.pallas_call(
        _proj_body,
        grid=(NB,),
        in_specs=[
            pl.BlockSpec((RB, H), lambda i: (i, 0)),
            pl.BlockSpec((H, H), lambda i: (0, 0)),
            pl.BlockSpec((1, H), lambda i: (0, 0)),
        ],
        out_specs=pl.BlockSpec((RB, H), lambda i: (i, 0)),
        out_shape=jax.ShapeDtypeStruct((N, H), jnp.float32),
    )(f_atoms, W_in, b_in2)


def _mlp_body(a0_ref, a1_ref, x_ref, w1_ref, b1_ref, w2_ref, b2_ref, ep_ref,
              h_ref, st_ref):
    i = pl.program_id(0)
    out = a0_ref[...] + a1_ref[...] + ep_ref[0, 0] * x_ref[...]
    h1 = jnp.maximum(
        jnp.dot(out, w1_ref[...], preferred_element_type=jnp.float32)
        + b1_ref[...],
        0.0,
    )
    h = (
        jnp.dot(h1, w2_ref[...], preferred_element_type=jnp.float32)
        + b2_ref[...]
    )
    h_ref[...] = h

    @pl.when(i == 0)
    def _():
        st_ref[...] = jnp.zeros_like(st_ref)

    st_ref[0:1, :] += jnp.sum(h, axis=0, keepdims=True)
    st_ref[1:2, :] += jnp.sum(h * h, axis=0, keepdims=True)


def _mlp(agg0, agg1, x, W1l, b1l2, W2l, b2l2, ep11):
    return pl.pallas_call(
        _mlp_body,
        grid=(NB,),
        in_specs=[
            pl.BlockSpec((RB, H), lambda i: (i, 0)),
            pl.BlockSpec((RB, H), lambda i: (i, 0)),
            pl.BlockSpec((RB, H), lambda i: (i, 0)),
            pl.BlockSpec((H, H), lambda i: (0, 0)),
            pl.BlockSpec((1, H), lambda i: (0, 0)),
            pl.BlockSpec((H, H), lambda i: (0, 0)),
            pl.BlockSpec((1, H), lambda i: (0, 0)),
            pl.BlockSpec(memory_space=pltpu.SMEM),
        ],
        out_specs=[
            pl.BlockSpec((RB, H), lambda i: (i, 0)),
            pl.BlockSpec((8, H), lambda i: (0, 0)),
        ],
        out_shape=[
            jax.ShapeDtypeStruct((N, H), jnp.float32),
            jax.ShapeDtypeStruct((8, H), jnp.float32),
        ],
    )(agg0, agg1, x, W1l, b1l2, W2l, b2l2, ep11)


def _bn_body(h_ref, st_ref, g_ref, be_ref, x_ref):
    mu = st_ref[0:1, :] * (1.0 / N)
    var = st_ref[1:2, :] * (1.0 / N) - mu * mu
    inv = lax.rsqrt(var + BN_EPS_)
    x_ref[...] = jnp.maximum(
        (h_ref[...] - mu) * (inv * g_ref[...]) + be_ref[...], 0.0
    )


def _bn_relu(h, stats, gl2, bl2):
    return pl.pallas_call(
        _bn_body,
        grid=(NB,),
        in_specs=[
            pl.BlockSpec((RB, H), lambda i: (i, 0)),
            pl.BlockSpec((8, H), lambda i: (0, 0)),
            pl.BlockSpec((1, H), lambda i: (0, 0)),
            pl.BlockSpec((1, H), lambda i: (0, 0)),
        ],
        out_specs=pl.BlockSpec((RB, H), lambda i: (i, 0)),
        out_shape=jax.ShapeDtypeStruct((N, H), jnp.float32),
    )(h, stats, gl2, bl2)


def _pool_body(x_ref, gid_ref, mol_ref, sums_ref, cnts_ref):
    i = pl.program_id(0)

    @pl.when(i == 0)
    def _():
        sums_ref[...] = jnp.zeros_like(sums_ref)
        cnts_ref[...] = jnp.zeros_like(cnts_ref)
        mol_ref[...] = jnp.zeros_like(mol_ref)

    gid = gid_ref[0, 0, :]
    oh = (
        gid[:, None]
        == lax.broadcasted_iota(jnp.int32, (RB, G), 1)
    ).astype(jnp.float32)
    x = x_ref[...]
    sums_ref[...] += lax.dot_general(
        oh, x, (((0,), (0,)), ((), ())), preferred_element_type=jnp.float32
    )
    cnts_ref[...] += lax.dot_general(
        oh, jnp.ones((RB, H), jnp.float32), (((0,), (0,)), ((), ())),
        preferred_element_type=jnp.float32,
    )

    @pl.when(i == NB - 1)
    def _():
        mol_ref[...] = sums_ref[...] / jnp.maximum(cnts_ref[...], 1.0)


def _pool(x, gid3):
    return pl.pallas_call(
        _pool_body,
        grid=(NB,),
        in_specs=[
            pl.BlockSpec((RB, H), lambda i: (i, 0)),
            pl.BlockSpec((1, 1, RB), lambda i: (i, 0, 0)),
        ],
        out_specs=pl.BlockSpec((G, H), lambda i: (0, 0)),
        out_shape=jax.ShapeDtypeStruct((G, H), jnp.float32),
        scratch_shapes=[
            pltpu.VMEM((G, H), jnp.float32),
            pltpu.VMEM((G, H), jnp.float32),
        ],
    )(x, gid3)


def kernel(f_atoms, edge_index, graph_ids, W_in, b_in, W1, b1, W2, b2,
           gamma, beta, eps):
    src = edge_index[0].astype(jnp.int32)
    tgt = edge_index[1].astype(jnp.int32)
    npad = E_PAD - E
    src3 = jnp.concatenate([src, jnp.zeros((npad,), jnp.int32)]).reshape(
        NT, NCH, CHUNK
    )
    tgt1 = jnp.concatenate([tgt, jnp.full((npad,), N, jnp.int32)])
    gid3 = graph_ids.astype(jnp.int32).reshape(NB, 1, RB)

    x = _proj(f_atoms, W_in, b_in.reshape(1, H))
    for l in range(DEPTH_):
        partials = _sc_scatter(x, src3, tgt1)
        h, stats = _mlp(
            partials[0], partials[1], x,
            W1[l], b1[l].reshape(1, H), W2[l], b2[l].reshape(1, H),
            (1.0 + eps[l]).reshape(1, 1),
        )
        x = _bn_relu(h, stats, gamma[l].reshape(1, H), beta[l].reshape(1, H))
    return _pool(x, gid3)
